# baseline (speedup 1.0000x reference)
import functools
import jax
import jax.numpy as jnp
from jax.experimental import pallas as pl

NODE = 128
EDGE = 16
HID = 256
GLOB = 64
HEADS = 5


def _mm_kernel(a_ref, w_ref, o_ref):
    o_ref[...] = jnp.dot(a_ref[...], w_ref[...],
                         preferred_element_type=jnp.float32)


@functools.partial(jax.jit, static_argnames=("bm",))
def _mm(a, w, bm):
    M, K = a.shape
    _, N = w.shape
    grid = (M // bm,)
    return pl.pallas_call(
        _mm_kernel,
        grid=grid,
        in_specs=[
            pl.BlockSpec((bm, K), lambda i: (i, 0)),
            pl.BlockSpec((K, N), lambda i: (0, 0)),
        ],
        out_specs=pl.BlockSpec((bm, N), lambda i: (i, 0)),
        out_shape=jax.ShapeDtypeStruct((M, N), jnp.float32),
    )(a, w)


def _layer_norm(h, g, b):
    m = jnp.mean(h, axis=-1, keepdims=True)
    v = jnp.mean((h - m) ** 2, axis=-1, keepdims=True)
    return (h - m) / jnp.sqrt(v + 1e-5) * g + b


def kernel(x, edge_index, edge_attr, u, batch, params):
    row, col = edge_index[0], edge_index[1]
    n = x.shape[0]
    b = u.shape[0]
    for p in params['layers']:
        Wg = p['gamma_W']
        Wb = p['beta_W']
        ug = u @ Wg[2 * NODE:]
        ub = u @ Wb[2 * NODE:]
        nW = jnp.concatenate([Wg[:NODE], Wg[NODE:2 * NODE],
                              Wb[:NODE], Wb[NODE:2 * NODE]], axis=1)
        nt = _mm(x, nW, 2000)
        ga = nt[:, :EDGE] + ug[batch] + p['gamma_b']
        gb = nt[:, EDGE:2 * EDGE]
        ba = nt[:, 2 * EDGE:3 * EDGE] + ub[batch] + p['beta_b']
        bb = nt[:, 3 * EDGE:]
        gamma = jax.nn.sigmoid(ga[row] + gb[col])
        beta = ba[row] + bb[col]
        gea = gamma * edge_attr
        edge_attr = gea + beta
        xl = _mm(x, p['Wl'], 2000).reshape(n, HEADS, NODE)
        xr = _mm(x, p['Wr'], 2000).reshape(n, HEADS, NODE)
        baWe = _mm(ba, p['We'], 2000).reshape(n, HEADS, NODE)
        bbWe = _mm(bb, p['We'], 2000).reshape(n, HEADS, NODE)
        bconst = (p['beta_b'] @ p['We']).reshape(HEADS, NODE)
        XLp = xl + baWe
        XRp = xr + bbWe + bconst[None]
        EP = _mm(gea, p['We'], 4000).reshape(-1, HEADS, NODE)
        z = jax.nn.leaky_relu(XLp[row] + XRp[col] + EP, 0.2)
        score = jnp.sum(z * p['att'][None, :, :], axis=-1)
        m = jnp.max(score)
        ex = jnp.exp(score - m)
        den = jax.ops.segment_sum(ex, col, num_segments=n)
        num = jax.ops.segment_sum(xl[row] * ex[:, :, None], col,
                                  num_segments=n)
        out = num / (den[:, :, None] + 1e-16)
        attn = out.mean(axis=1) + p['gat_b']
        h = _layer_norm(x + attn, p['ln1_g'], p['ln1_b'])
        hm = jax.nn.selu(_mm(h, p['mlp_W1'], 2000) + p['mlp_b1'])
        hm = _layer_norm(hm, p['mlp_ln_g'], p['mlp_ln_b'])
        hm = _mm(hm, p['mlp_W2'], 2000) + p['mlp_b2']
        x = _layer_norm(h + hm, p['ln2_g'], p['ln2_b'])
        ones = jnp.ones((n, 1), x.dtype)
        cnt = jnp.maximum(jax.ops.segment_sum(ones, batch, num_segments=b), 1.0)
        s = jax.ops.segment_sum(x, batch, num_segments=b)
        mean = s / cnt
        s2 = jax.ops.segment_sum(x * x, batch, num_segments=b)
        var = s2 / cnt - mean ** 2
        std = jnp.sqrt(jax.nn.relu(var) + 1e-5)
        mx = jax.ops.segment_max(x, batch, num_segments=b)
        mn = jax.ops.segment_min(x, batch, num_segments=b)
        aggr = jnp.concatenate([mean, std, mx, mn], axis=1)
        g = jnp.concatenate([u, aggr], axis=1)
        g = jax.nn.selu(g @ p['g_W1'] + p['g_b1'])
        g = _layer_norm(g, p['g_ln_g'], p['g_ln_b'])
        u = g @ p['g_W2'] + p['g_b2']
    return u


# speedup vs baseline: 3.6287x; 3.6287x over previous
import functools
import jax
import jax.numpy as jnp
from jax import lax
from jax.experimental import pallas as pl
from jax.experimental.pallas import tpu as pltpu
from jax.experimental.pallas import tpu_sc as plsc

N = 10000
NPAD = 10112
NP5 = NPAD * 5
E = 160000
EPAD = 163840
NODE = 128
EDGE = 16
HID = 256
GLOB = 64
HEADS = 5
NC = 2
NS = 16
NW = NC * NS
EW = EPAD // NW
C = 128
NCH = EW // C
AGG = 128

_MESH = plsc.VectorSubcoreMesh(core_axis_name="c", subcore_axis_name="s",
                               num_cores=NC, num_subcores=NS)


_GDN = lax.GatherDimensionNumbers(offset_dims=(), collapsed_slice_dims=(0,),
                                  start_index_map=(0,))


def _permute(v, idx):
    return lax.gather(v, idx[:, None], _GDN, (1,),
                      mode=lax.GatherScatterMode.PROMISE_IN_BOUNDS)


def _butterfly(v, op):
    lanes = lax.iota(jnp.int32, 16)
    for sh in (8, 4, 2, 1):
        v = op(v, _permute(v, jnp.bitwise_xor(lanes, sh)))
    return v


def _mm_kernel(a_ref, w_ref, o_ref):
    o_ref[...] = jnp.dot(a_ref[...], w_ref[...],
                         preferred_element_type=jnp.float32)


@functools.partial(jax.jit, static_argnames=("bm",))
def _mm(a, w, bm):
    M, K = a.shape
    _, Nn = w.shape
    return pl.pallas_call(
        _mm_kernel,
        grid=(M // bm,),
        in_specs=[
            pl.BlockSpec((bm, K), lambda i: (i, 0)),
            pl.BlockSpec((K, Nn), lambda i: (0, 0)),
        ],
        out_specs=pl.BlockSpec((bm, Nn), lambda i: (i, 0)),
        out_shape=jax.ShapeDtypeStruct((M, Nn), jnp.float32),
    )(a, w)


def _ep_kernel(gea_ref, we_ref, o_ref):
    g = gea_ref[...]
    w = we_ref[...]
    for h in range(HEADS):
        o_ref[h] = jnp.dot(g, w[:, h * NODE:(h + 1) * NODE],
                           preferred_element_type=jnp.float32)


@jax.jit
def _ep(gea, we):
    be = 4096
    return pl.pallas_call(
        _ep_kernel,
        grid=(EPAD // be,),
        in_specs=[
            pl.BlockSpec((be, EDGE), lambda i: (i, 0)),
            pl.BlockSpec((EDGE, HEADS * NODE), lambda i: (0, 0)),
        ],
        out_specs=pl.BlockSpec((HEADS, be, NODE), lambda i: (0, i, 0)),
        out_shape=jax.ShapeDtypeStruct((HEADS, EPAD, NODE), jnp.float32),
    )(gea, we)


@functools.partial(
    pl.kernel,
    out_type=(jax.ShapeDtypeStruct((EPAD, EDGE), jnp.float32),
              jax.ShapeDtypeStruct((EPAD, EDGE), jnp.float32)),
    mesh=_MESH,
    scratch_types=[
        pltpu.VMEM((C,), jnp.int32),
        pltpu.VMEM((C,), jnp.int32),
        pltpu.VMEM((C, 128), jnp.float32),
        pltpu.VMEM((C, 128), jnp.float32),
        pltpu.VMEM((C, EDGE), jnp.float32),
        pltpu.VMEM((C, EDGE), jnp.float32),
        pltpu.VMEM((C, EDGE), jnp.float32),
        pltpu.SemaphoreType.DMA,
    ],
)
def _sc_film(row_hbm, col_hbm, nt_hbm, ea_hbm,
             ean_hbm, gea_hbm,
             idxr, idxc, abuf, bbuf, eabuf, eanbuf, geabuf, sem):
    wid = lax.axis_index("c") * NS + lax.axis_index("s")

    def chunk(ci, _):
        base = wid * EW + ci * C
        pltpu.sync_copy(row_hbm.at[pl.ds(base, C)], idxr)
        pltpu.sync_copy(col_hbm.at[pl.ds(base, C)], idxc)
        pltpu.async_copy(nt_hbm.at[idxr], abuf, sem).wait()
        pltpu.async_copy(nt_hbm.at[idxc], bbuf, sem).wait()
        pltpu.sync_copy(ea_hbm.at[pl.ds(base, C)], eabuf)

        def edge(i, _):
            ag = abuf[i, 0:16]
            ab = abuf[i, 16:32]
            bg = bbuf[i, 32:48]
            bb = bbuf[i, 48:64]
            gamma = 1.0 / (1.0 + jnp.exp(-(ag + bg)))
            ge = gamma * eabuf[i, :]
            geabuf[i, :] = ge
            eanbuf[i, :] = ge + ab + bb
            return 0

        lax.fori_loop(0, C, edge, 0)
        pltpu.sync_copy(eanbuf, ean_hbm.at[pl.ds(base, C)])
        pltpu.sync_copy(geabuf, gea_hbm.at[pl.ds(base, C)])
        return 0

    lax.fori_loop(0, NCH, chunk, 0)


@functools.partial(
    pl.kernel,
    out_type=tuple([jax.ShapeDtypeStruct((EPAD,), jnp.float32)] * HEADS
                   + [jax.ShapeDtypeStruct((NW, 8, 128), jnp.float32)]),
    mesh=_MESH,
    scratch_types=[
        pltpu.VMEM((C,), jnp.int32),
        pltpu.VMEM((C,), jnp.int32),
        pltpu.VMEM((C,), jnp.int32),
        pltpu.VMEM((C,), jnp.int32),
        pltpu.VMEM((C, NODE), jnp.float32),
        pltpu.VMEM((C, NODE), jnp.float32),
        pltpu.VMEM((C, NODE), jnp.float32),
        pltpu.VMEM((C,), jnp.float32),
        pltpu.VMEM((HEADS, NODE), jnp.float32),
        pltpu.VMEM((8, 128), jnp.float32),
        pltpu.SemaphoreType.DMA,
    ],
)
def _sc_score(row5_hbm, col5_hbm, xlp_hbm, xrp_hbm, ep_hbm, att_hbm,
              sc0, sc1, sc2, sc3, sc4, max_hbm,
              idxl, idxr_, r5buf, c5buf, xlb, xrb, epb, sbuf, attb, mbuf, sem):
    wid = lax.axis_index("c") * NS + lax.axis_index("s")
    score_hbms = [sc0, sc1, sc2, sc3, sc4]
    pltpu.sync_copy(att_hbm, attb)

    for h in range(HEADS):
        att_v = [attb[h, pl.ds(16 * j, 16)] for j in range(8)]

        def chunk(ci, m_h):
            base = wid * EW + ci * C
            pltpu.sync_copy(row5_hbm.at[pl.ds(base, C)], r5buf)
            pltpu.sync_copy(col5_hbm.at[pl.ds(base, C)], c5buf)
            for j in range(C // 16):
                sl = pl.ds(16 * j, 16)
                idxl[sl] = r5buf[sl] + h
                idxr_[sl] = c5buf[sl] + h
            pltpu.async_copy(xlp_hbm.at[idxl], xlb, sem).wait()
            pltpu.async_copy(xrp_hbm.at[idxr_], xrb, sem).wait()
            pltpu.sync_copy(ep_hbm.at[h, pl.ds(base, C)], epb)

            lanes = lax.iota(jnp.int32, 16)

            def group(g, m_carry):
                def edge(k, sv):
                    i = g * 16 + k
                    acc = None
                    for j in range(8):
                        sl = pl.ds(16 * j, 16)
                        v = xlb[i, sl] + xrb[i, sl] + epb[i, sl]
                        v = 0.6 * v + 0.4 * jnp.abs(v)
                        t = v * att_v[j]
                        acc = t if acc is None else acc + t
                    s = _butterfly(acc, jnp.add)
                    return jnp.where(lanes == k, s, sv)

                sv = lax.fori_loop(0, 16, edge,
                                   jnp.full((16,), -3.0e38, jnp.float32))
                sbuf[pl.ds(g * 16, 16)] = sv
                return jnp.maximum(m_carry, sv)

            m_h = lax.fori_loop(0, C // 16, group, m_h)
            pltpu.sync_copy(sbuf, score_hbms[h].at[pl.ds(base, C)])
            return m_h

        m_h = lax.fori_loop(0, NCH, chunk,
                            jnp.full((16,), -3.0e38, jnp.float32))
        mbuf[h, 0:16] = m_h

    pltpu.sync_copy(mbuf, max_hbm.at[wid])


@functools.partial(
    pl.kernel,
    out_type=jax.ShapeDtypeStruct((HEADS + 1, NC, NPAD, AGG), jnp.float32),
    mesh=_MESH,
    scratch_types=[
        pltpu.VMEM((C,), jnp.int32),
        pltpu.VMEM((C,), jnp.int32),
        pltpu.VMEM((C,), jnp.int32),
        pltpu.VMEM((C, AGG), jnp.float32),
        pltpu.VMEM((C,), jnp.float32),
        pltpu.VMEM((HEADS, C), jnp.float32),
        pltpu.VMEM((8, 128), jnp.float32),
        pltpu.VMEM((C, AGG), jnp.float32),
        pltpu.VMEM_SHARED((NPAD, AGG), jnp.float32),
        pltpu.SemaphoreType.DMA,
    ],
)
def _sc_agg(row5_hbm, col_hbm, sc0, sc1, sc2, sc3, sc4, max_hbm, xla_hbm,
            parts_hbm,
            r5buf, idxl, idxc, rbuf, sbuf, exbuf, mxbuf, zbuf,
            accum, sem):
    score_hbms = [sc0, sc1, sc2, sc3, sc4]
    cid = lax.axis_index("c")
    sid = lax.axis_index("s")
    wid = cid * NS + sid
    lanes = lax.iota(jnp.int32, 16)

    pltpu.sync_copy(max_hbm, mxbuf)
    m_sc = [mxbuf[h, 0:16] for h in range(HEADS)]

    def zrow(i, _):
        for j in range(AGG // 16):
            zbuf[i, pl.ds(16 * j, 16)] = jnp.zeros((16,), jnp.float32)
        return 0

    lax.fori_loop(0, C, zrow, 0)

    rows_per_tile = NPAD // NS
    rem = rows_per_tile - (rows_per_tile // C) * C

    def _zero_accum():
        def zchunk(i, _):
            pltpu.sync_copy(zbuf,
                            accum.at[pl.ds(sid * rows_per_tile + i * C, C)])
            return 0

        lax.fori_loop(0, rows_per_tile // C, zchunk, 0)
        pltpu.sync_copy(
            zbuf.at[pl.ds(0, rem)],
            accum.at[pl.ds(sid * rows_per_tile + (rows_per_tile // C) * C,
                           rem)])

    def _dump(pi):
        pltpu.sync_copy(accum.at[pl.ds(sid * rows_per_tile, rows_per_tile)],
                        parts_hbm.at[pi, cid, pl.ds(sid * rows_per_tile,
                                                    rows_per_tile)])

    for h in range(HEADS):
        _zero_accum()
        plsc.subcore_barrier()
        m_h = m_sc[h]

        def chunk(ci, _):
            base = wid * EW + ci * C
            pltpu.sync_copy(row5_hbm.at[pl.ds(base, C)], r5buf)
            pltpu.sync_copy(col_hbm.at[pl.ds(base, C)], idxc)
            for j in range(C // 16):
                sl = pl.ds(16 * j, 16)
                idxl[sl] = r5buf[sl] + h
            pltpu.async_copy(xla_hbm.at[idxl], rbuf, sem).wait()
            pltpu.sync_copy(score_hbms[h].at[pl.ds(base, C)], sbuf)
            for j in range(C // 16):
                sl = pl.ds(16 * j, 16)
                exbuf[0, sl] = jnp.exp(sbuf[sl] - m_h)

            def group(g, _):
                ex_g = exbuf[0, pl.ds(g * 16, 16)]

                def lane(k, _):
                    i = g * 16 + k
                    s_v = _permute(ex_g, jnp.full((16,), k, jnp.int32))
                    for j in range(AGG // 16):
                        sl = pl.ds(16 * j, 16)
                        rbuf[i, sl] = rbuf[i, sl] * s_v
                    return 0

                lax.fori_loop(0, 16, lane, 0)
                return 0

            lax.fori_loop(0, C // 16, group, 0)
            pltpu.sync_copy(rbuf, accum.at[idxc], add=True)
            return 0

        lax.fori_loop(0, NCH, chunk, 0)
        plsc.subcore_barrier()
        _dump(h)
        plsc.subcore_barrier()

    _zero_accum()
    plsc.subcore_barrier()

    def zstage(i, _):
        for j in range(AGG // 16):
            rbuf[i, pl.ds(16 * j, 16)] = jnp.zeros((16,), jnp.float32)
        return 0

    lax.fori_loop(0, C, zstage, 0)

    def dchunk(ci, _):
        base = wid * EW + ci * C
        pltpu.sync_copy(col_hbm.at[pl.ds(base, C)], idxc)
        for h in range(HEADS):
            pltpu.sync_copy(score_hbms[h].at[pl.ds(base, C)], sbuf)
            for j in range(C // 16):
                sl = pl.ds(16 * j, 16)
                exbuf[h, sl] = jnp.exp(sbuf[sl] - m_sc[h])

        def group(g, _):
            ex_gs = [exbuf[h, pl.ds(g * 16, 16)] for h in range(HEADS)]

            def lane(k, _):
                i = g * 16 + k
                kk = jnp.full((16,), k, jnp.int32)
                v = jnp.zeros((16,), jnp.float32)
                for h in range(HEADS):
                    v = jnp.where(lanes == h, _permute(ex_gs[h], kk), v)
                rbuf[i, 0:16] = v
                return 0

            lax.fori_loop(0, 16, lane, 0)
            return 0

        lax.fori_loop(0, C // 16, group, 0)
        pltpu.sync_copy(rbuf, accum.at[idxc], add=True)
        return 0

    lax.fori_loop(0, NCH, dchunk, 0)
    plsc.subcore_barrier()
    _dump(HEADS)


def _layer_norm(h, g, b):
    m = jnp.mean(h, axis=-1, keepdims=True)
    v = jnp.mean((h - m) ** 2, axis=-1, keepdims=True)
    return (h - m) / jnp.sqrt(v + 1e-5) * g + b


def kernel(x, edge_index, edge_attr, u, batch, params):
    row, col = edge_index[0], edge_index[1]
    n = x.shape[0]
    b = u.shape[0]
    pad = EPAD - E
    rowp = jnp.concatenate([row, jnp.zeros((pad,), jnp.int32)])
    colp = jnp.concatenate([col, jnp.full((pad,), N, jnp.int32)])
    row5 = rowp * 5
    col5 = colp * 5
    ea = jnp.concatenate([edge_attr, jnp.zeros((pad, EDGE), jnp.float32)])

    for p in params['layers']:
        Wg = p['gamma_W']
        Wb = p['beta_W']
        ug = u @ Wg[2 * NODE:]
        ub = u @ Wb[2 * NODE:]
        nW = jnp.concatenate([Wg[:NODE], Wg[NODE:2 * NODE],
                              Wb[:NODE], Wb[NODE:2 * NODE]], axis=1)
        nt = _mm(x, nW, 2000)
        ga = nt[:, :EDGE] + ug[batch] + p['gamma_b']
        gb = nt[:, EDGE:2 * EDGE]
        ba = nt[:, 2 * EDGE:3 * EDGE] + ub[batch] + p['beta_b']
        bb = nt[:, 3 * EDGE:]
        NT = jnp.pad(jnp.concatenate([ga, ba, gb, bb], axis=1),
                     ((0, NPAD - N), (0, NODE - 4 * EDGE)))

        ea, gea = _sc_film(rowp, colp, NT, ea)

        xl = _mm(x, p['Wl'], 2000)
        xr = _mm(x, p['Wr'], 2000)
        baWe = _mm(ba, p['We'], 2000)
        bbWe = _mm(bb, p['We'], 2000)
        bconst = p['beta_b'] @ p['We']
        XLp = jnp.pad((xl + baWe).reshape(N * 5, NODE),
                      ((0, NP5 - N * 5), (0, 0)))
        XRp = jnp.pad((xr + bbWe + bconst[None]).reshape(N * 5, NODE),
                      ((0, NP5 - N * 5), (0, 0)))
        XLA = jnp.pad(xl.reshape(N * 5, NODE), ((0, NP5 - N * 5), (0, 0)))

        EPt = _ep(gea, p['We'])
        s0, s1, s2, s3, s4, maxes = _sc_score(row5, col5, XLp, XRp, EPt,
                                              p['att'])
        m_sc = jnp.max(maxes[:, :HEADS, :16], axis=(0, 2))
        m_in = jnp.broadcast_to(
            jnp.pad(m_sc, (0, 3))[:, None], (8, 128))
        parts = _sc_agg(row5, colp, s0, s1, s2, s3, s4, m_in, XLA)

        agg = parts[:, 0] + parts[:, 1]
        num = agg[:HEADS, :N, :]
        den = jnp.transpose(agg[HEADS, :N, :HEADS])
        out = num / (den[:, :, None] + 1e-16)
        attn = out.mean(axis=0) + p['gat_b']

        h = _layer_norm(x + attn, p['ln1_g'], p['ln1_b'])
        hm = jax.nn.selu(_mm(h, p['mlp_W1'], 2000) + p['mlp_b1'])
        hm = _layer_norm(hm, p['mlp_ln_g'], p['mlp_ln_b'])
        hm = _mm(hm, p['mlp_W2'], 2000) + p['mlp_b2']
        x = _layer_norm(h + hm, p['ln2_g'], p['ln2_b'])

        ones = jnp.ones((n, 1), x.dtype)
        cnt = jnp.maximum(jax.ops.segment_sum(ones, batch, num_segments=b), 1.0)
        s = jax.ops.segment_sum(x, batch, num_segments=b)
        mean = s / cnt
        s2 = jax.ops.segment_sum(x * x, batch, num_segments=b)
        var = s2 / cnt - mean ** 2
        std = jnp.sqrt(jax.nn.relu(var) + 1e-5)
        mx = jax.ops.segment_max(x, batch, num_segments=b)
        mn = jax.ops.segment_min(x, batch, num_segments=b)
        aggr = jnp.concatenate([mean, std, mx, mn], axis=1)
        g = jnp.concatenate([u, aggr], axis=1)
        g = jax.nn.selu(g @ p['g_W1'] + p['g_b1'])
        g = _layer_norm(g, p['g_ln_g'], p['g_ln_b'])
        u = g @ p['g_W2'] + p['g_b2']
    return u


# speedup vs baseline: 4.6187x; 1.2728x over previous
import functools
import jax
import jax.numpy as jnp
from jax import lax
from jax.experimental import pallas as pl
from jax.experimental.pallas import tpu as pltpu
from jax.experimental.pallas import tpu_sc as plsc

N = 10000
NPAD = 10112
NP5 = NPAD * 5
E = 160000
EPAD = 163840
NODE = 128
EDGE = 16
HID = 256
GLOB = 64
HEADS = 5
NC = 2
NS = 16
NW = NC * NS
EW = EPAD // NW
C = 128
NCH = EW // C
AGG = 128

_MESH = plsc.VectorSubcoreMesh(core_axis_name="c", subcore_axis_name="s",
                               num_cores=NC, num_subcores=NS)


_GDN = lax.GatherDimensionNumbers(offset_dims=(), collapsed_slice_dims=(0,),
                                  start_index_map=(0,))


def _permute(v, idx):
    return lax.gather(v, idx[:, None], _GDN, (1,),
                      mode=lax.GatherScatterMode.PROMISE_IN_BOUNDS)


def _butterfly(v, op):
    lanes = lax.iota(jnp.int32, 16)
    for sh in (8, 4, 2, 1):
        v = op(v, _permute(v, jnp.bitwise_xor(lanes, sh)))
    return v


def _mm_kernel(a_ref, w_ref, o_ref):
    o_ref[...] = jnp.dot(a_ref[...], w_ref[...],
                         preferred_element_type=jnp.float32)


@functools.partial(jax.jit, static_argnames=("bm",))
def _mm(a, w, bm):
    M, K = a.shape
    _, Nn = w.shape
    return pl.pallas_call(
        _mm_kernel,
        grid=(M // bm,),
        in_specs=[
            pl.BlockSpec((bm, K), lambda i: (i, 0)),
            pl.BlockSpec((K, Nn), lambda i: (0, 0)),
        ],
        out_specs=pl.BlockSpec((bm, Nn), lambda i: (i, 0)),
        out_shape=jax.ShapeDtypeStruct((M, Nn), jnp.float32),
    )(a, w)


def _ep_kernel(gea_ref, we_ref, o_ref):
    g = gea_ref[...]
    w = we_ref[...]
    for h in range(HEADS):
        o_ref[h] = jnp.dot(g, w[:, h * NODE:(h + 1) * NODE],
                           preferred_element_type=jnp.float32)


@jax.jit
def _ep(gea, we):
    be = 4096
    return pl.pallas_call(
        _ep_kernel,
        grid=(EPAD // be,),
        in_specs=[
            pl.BlockSpec((be, EDGE), lambda i: (i, 0)),
            pl.BlockSpec((EDGE, HEADS * NODE), lambda i: (0, 0)),
        ],
        out_specs=pl.BlockSpec((HEADS, be, NODE), lambda i: (0, i, 0)),
        out_shape=jax.ShapeDtypeStruct((HEADS, EPAD, NODE), jnp.float32),
    )(gea, we)


@functools.partial(
    pl.kernel,
    out_type=(jax.ShapeDtypeStruct((EPAD, EDGE), jnp.float32),
              jax.ShapeDtypeStruct((EPAD, EDGE), jnp.float32)),
    mesh=_MESH,
    scratch_types=[
        pltpu.VMEM((C,), jnp.int32),
        pltpu.VMEM((C,), jnp.int32),
        pltpu.VMEM((C, 128), jnp.float32),
        pltpu.VMEM((C, 128), jnp.float32),
        pltpu.VMEM((C, EDGE), jnp.float32),
        pltpu.VMEM((C, EDGE), jnp.float32),
        pltpu.VMEM((C, EDGE), jnp.float32),
        pltpu.SemaphoreType.DMA,
    ],
)
def _sc_film(row_hbm, col_hbm, nt_hbm, ea_hbm,
             ean_hbm, gea_hbm,
             idxr, idxc, abuf, bbuf, eabuf, eanbuf, geabuf, sem):
    wid = lax.axis_index("c") * NS + lax.axis_index("s")

    def chunk(ci, _):
        base = wid * EW + ci * C
        pltpu.sync_copy(row_hbm.at[pl.ds(base, C)], idxr)
        pltpu.sync_copy(col_hbm.at[pl.ds(base, C)], idxc)
        pltpu.async_copy(nt_hbm.at[idxr], abuf, sem).wait()
        pltpu.async_copy(nt_hbm.at[idxc], bbuf, sem).wait()
        pltpu.sync_copy(ea_hbm.at[pl.ds(base, C)], eabuf)

        def edge(i, _):
            ag = abuf[i, 0:16]
            ab = abuf[i, 16:32]
            bg = bbuf[i, 32:48]
            bb = bbuf[i, 48:64]
            gamma = 1.0 / (1.0 + jnp.exp(-(ag + bg)))
            ge = gamma * eabuf[i, :]
            geabuf[i, :] = ge
            eanbuf[i, :] = ge + ab + bb
            return 0

        lax.fori_loop(0, C, edge, 0)
        pltpu.sync_copy(eanbuf, ean_hbm.at[pl.ds(base, C)])
        pltpu.sync_copy(geabuf, gea_hbm.at[pl.ds(base, C)])
        return 0

    lax.fori_loop(0, NCH, chunk, 0)


@functools.partial(
    pl.kernel,
    out_type=tuple([jax.ShapeDtypeStruct((EPAD,), jnp.float32)] * HEADS
                   + [jax.ShapeDtypeStruct((NW, 8, 128), jnp.float32)]),
    mesh=_MESH,
    scratch_types=[
        pltpu.VMEM((2, C), jnp.int32),
        pltpu.VMEM((2, C), jnp.int32),
        pltpu.VMEM((C,), jnp.int32),
        pltpu.VMEM((C,), jnp.int32),
        pltpu.VMEM((2, C, NODE), jnp.float32),
        pltpu.VMEM((2, C, NODE), jnp.float32),
        pltpu.VMEM((2, C, NODE), jnp.float32),
        pltpu.VMEM((C,), jnp.float32),
        pltpu.VMEM((HEADS, NODE), jnp.float32),
        pltpu.VMEM((8, 128), jnp.float32),
        pltpu.SemaphoreType.DMA,
        pltpu.SemaphoreType.DMA,
    ],
)
def _sc_score(row5_hbm, col5_hbm, xlp_hbm, xrp_hbm, ep_hbm, att_hbm,
              sc0, sc1, sc2, sc3, sc4, max_hbm,
              idxl, idxr_, r5buf, c5buf, xlb, xrb, epb, sbuf, attb, mbuf,
              sem0, sem1):
    wid = lax.axis_index("c") * NS + lax.axis_index("s")
    score_hbms = [sc0, sc1, sc2, sc3, sc4]
    sems = [sem0, sem1]
    pltpu.sync_copy(att_hbm, attb)
    lanes = lax.iota(jnp.int32, 16)

    for h in range(HEADS):
        att_v = [attb[h, pl.ds(16 * j, 16)] for j in range(8)]

        def issue(ci, b):
            base = wid * EW + ci * C
            pltpu.sync_copy(row5_hbm.at[pl.ds(base, C)], r5buf)
            pltpu.sync_copy(col5_hbm.at[pl.ds(base, C)], c5buf)
            for j in range(C // 16):
                sl = pl.ds(16 * j, 16)
                idxl[b, sl] = r5buf[sl] + h
                idxr_[b, sl] = c5buf[sl] + h
            pltpu.async_copy(xlp_hbm.at[idxl.at[b]], xlb.at[b], sems[b])
            pltpu.async_copy(xrp_hbm.at[idxr_.at[b]], xrb.at[b], sems[b])
            pltpu.async_copy(ep_hbm.at[h, pl.ds(base, C)], epb.at[b], sems[b])

        def drain(b):
            pltpu.make_async_copy(xlp_hbm.at[idxl.at[b]], xlb.at[b],
                                  sems[b]).wait()
            pltpu.make_async_copy(xrp_hbm.at[idxr_.at[b]], xrb.at[b],
                                  sems[b]).wait()
            pltpu.make_async_copy(ep_hbm.at[h, pl.ds(0, C)], epb.at[b],
                                  sems[b]).wait()

        def compute(ci, b, m_h):
            base = wid * EW + ci * C

            def group(g, m_carry):
                def edge(k, sv):
                    i = g * 16 + k
                    acc = None
                    for j in range(8):
                        sl = pl.ds(16 * j, 16)
                        v = xlb[b, i, sl] + xrb[b, i, sl] + epb[b, i, sl]
                        v = 0.6 * v + 0.4 * jnp.abs(v)
                        t = v * att_v[j]
                        acc = t if acc is None else acc + t
                    s = _butterfly(acc, jnp.add)
                    return jnp.where(lanes == k, s, sv)

                sv = lax.fori_loop(0, 16, edge,
                                   jnp.full((16,), -3.0e38, jnp.float32))
                sbuf[pl.ds(g * 16, 16)] = sv
                return jnp.maximum(m_carry, sv)

            m_h = lax.fori_loop(0, C // 16, group, m_h)
            pltpu.sync_copy(sbuf, score_hbms[h].at[pl.ds(base, C)])
            return m_h

        issue(0, 0)

        def chunk2(ci2, m_h):
            issue(2 * ci2 + 1, 1)
            drain(0)
            m_h = compute(2 * ci2, 0, m_h)
            issue(lax.rem(2 * ci2 + 2, NCH), 0)
            drain(1)
            m_h = compute(2 * ci2 + 1, 1, m_h)
            return m_h

        m_h = lax.fori_loop(0, NCH // 2, chunk2,
                            jnp.full((16,), -3.0e38, jnp.float32))
        drain(0)
        mbuf[h, 0:16] = m_h

    pltpu.sync_copy(mbuf, max_hbm.at[wid])


@functools.partial(
    pl.kernel,
    out_type=jax.ShapeDtypeStruct((HEADS + 1, NC, NPAD, AGG), jnp.float32),
    mesh=_MESH,
    scratch_types=[
        pltpu.VMEM((C,), jnp.int32),
        pltpu.VMEM((C,), jnp.int32),
        pltpu.VMEM((C,), jnp.int32),
        pltpu.VMEM((C, AGG), jnp.float32),
        pltpu.VMEM((C,), jnp.float32),
        pltpu.VMEM((HEADS, C), jnp.float32),
        pltpu.VMEM((8, 128), jnp.float32),
        pltpu.VMEM((C, AGG), jnp.float32),
        pltpu.VMEM_SHARED((NPAD, AGG), jnp.float32),
        pltpu.SemaphoreType.DMA,
    ],
)
def _sc_agg(row5_hbm, col_hbm, sc0, sc1, sc2, sc3, sc4, max_hbm, xla_hbm,
            parts_hbm,
            r5buf, idxl, idxc, rbuf, sbuf, exbuf, mxbuf, zbuf,
            accum, sem):
    score_hbms = [sc0, sc1, sc2, sc3, sc4]
    cid = lax.axis_index("c")
    sid = lax.axis_index("s")
    wid = cid * NS + sid
    lanes = lax.iota(jnp.int32, 16)

    pltpu.sync_copy(max_hbm, mxbuf)
    m_sc = [mxbuf[h, 0:16] for h in range(HEADS)]

    def zrow(i, _):
        for j in range(AGG // 16):
            zbuf[i, pl.ds(16 * j, 16)] = jnp.zeros((16,), jnp.float32)
        return 0

    lax.fori_loop(0, C, zrow, 0)

    rows_per_tile = NPAD // NS
    rem = rows_per_tile - (rows_per_tile // C) * C

    def _zero_accum():
        def zchunk(i, _):
            pltpu.sync_copy(zbuf,
                            accum.at[pl.ds(sid * rows_per_tile + i * C, C)])
            return 0

        lax.fori_loop(0, rows_per_tile // C, zchunk, 0)
        pltpu.sync_copy(
            zbuf.at[pl.ds(0, rem)],
            accum.at[pl.ds(sid * rows_per_tile + (rows_per_tile // C) * C,
                           rem)])

    def _dump(pi):
        pltpu.sync_copy(accum.at[pl.ds(sid * rows_per_tile, rows_per_tile)],
                        parts_hbm.at[pi, cid, pl.ds(sid * rows_per_tile,
                                                    rows_per_tile)])

    for h in range(HEADS):
        _zero_accum()
        plsc.subcore_barrier()
        m_h = m_sc[h]

        def chunk(ci, _):
            base = wid * EW + ci * C
            pltpu.sync_copy(row5_hbm.at[pl.ds(base, C)], r5buf)
            pltpu.sync_copy(col_hbm.at[pl.ds(base, C)], idxc)
            for j in range(C // 16):
                sl = pl.ds(16 * j, 16)
                idxl[sl] = r5buf[sl] + h
            pltpu.async_copy(xla_hbm.at[idxl], rbuf, sem).wait()
            pltpu.sync_copy(score_hbms[h].at[pl.ds(base, C)], sbuf)
            for j in range(C // 16):
                sl = pl.ds(16 * j, 16)
                exbuf[0, sl] = jnp.exp(sbuf[sl] - m_h)

            def group(g, _):
                ex_g = exbuf[0, pl.ds(g * 16, 16)]

                def lane(k, _):
                    i = g * 16 + k
                    s_v = _permute(ex_g, jnp.full((16,), k, jnp.int32))
                    for j in range(AGG // 16):
                        sl = pl.ds(16 * j, 16)
                        rbuf[i, sl] = rbuf[i, sl] * s_v
                    return 0

                lax.fori_loop(0, 16, lane, 0)
                return 0

            lax.fori_loop(0, C // 16, group, 0)
            pltpu.sync_copy(rbuf, accum.at[idxc], add=True)
            return 0

        lax.fori_loop(0, NCH, chunk, 0)
        plsc.subcore_barrier()
        _dump(h)
        plsc.subcore_barrier()

    _zero_accum()
    plsc.subcore_barrier()

    def zstage(i, _):
        for j in range(AGG // 16):
            rbuf[i, pl.ds(16 * j, 16)] = jnp.zeros((16,), jnp.float32)
        return 0

    lax.fori_loop(0, C, zstage, 0)

    def dchunk(ci, _):
        base = wid * EW + ci * C
        pltpu.sync_copy(col_hbm.at[pl.ds(base, C)], idxc)
        for h in range(HEADS):
            pltpu.sync_copy(score_hbms[h].at[pl.ds(base, C)], sbuf)
            for j in range(C // 16):
                sl = pl.ds(16 * j, 16)
                exbuf[h, sl] = jnp.exp(sbuf[sl] - m_sc[h])

        def group(g, _):
            ex_gs = [exbuf[h, pl.ds(g * 16, 16)] for h in range(HEADS)]

            def lane(k, _):
                i = g * 16 + k
                kk = jnp.full((16,), k, jnp.int32)
                v = jnp.zeros((16,), jnp.float32)
                for h in range(HEADS):
                    v = jnp.where(lanes == h, _permute(ex_gs[h], kk), v)
                rbuf[i, 0:16] = v
                return 0

            lax.fori_loop(0, 16, lane, 0)
            return 0

        lax.fori_loop(0, C // 16, group, 0)
        pltpu.sync_copy(rbuf, accum.at[idxc], add=True)
        return 0

    lax.fori_loop(0, NCH, dchunk, 0)
    plsc.subcore_barrier()
    _dump(HEADS)


def _layer_norm(h, g, b):
    m = jnp.mean(h, axis=-1, keepdims=True)
    v = jnp.mean((h - m) ** 2, axis=-1, keepdims=True)
    return (h - m) / jnp.sqrt(v + 1e-5) * g + b


def kernel(x, edge_index, edge_attr, u, batch, params):
    row, col = edge_index[0], edge_index[1]
    n = x.shape[0]
    b = u.shape[0]
    pad = EPAD - E
    rowp = jnp.concatenate([row, jnp.zeros((pad,), jnp.int32)])
    colp = jnp.concatenate([col, jnp.full((pad,), N, jnp.int32)])
    row5 = rowp * 5
    col5 = colp * 5
    ea = jnp.concatenate([edge_attr, jnp.zeros((pad, EDGE), jnp.float32)])

    for p in params['layers']:
        Wg = p['gamma_W']
        Wb = p['beta_W']
        ug = u @ Wg[2 * NODE:]
        ub = u @ Wb[2 * NODE:]
        nW = jnp.concatenate([Wg[:NODE], Wg[NODE:2 * NODE],
                              Wb[:NODE], Wb[NODE:2 * NODE]], axis=1)
        nt = _mm(x, nW, 2000)
        ga = nt[:, :EDGE] + ug[batch] + p['gamma_b']
        gb = nt[:, EDGE:2 * EDGE]
        ba = nt[:, 2 * EDGE:3 * EDGE] + ub[batch] + p['beta_b']
        bb = nt[:, 3 * EDGE:]
        NT = jnp.pad(jnp.concatenate([ga, ba, gb, bb], axis=1),
                     ((0, NPAD - N), (0, NODE - 4 * EDGE)))

        ea, gea = _sc_film(rowp, colp, NT, ea)

        xl = _mm(x, p['Wl'], 2000)
        xr = _mm(x, p['Wr'], 2000)
        baWe = _mm(ba, p['We'], 2000)
        bbWe = _mm(bb, p['We'], 2000)
        bconst = p['beta_b'] @ p['We']
        XLp = jnp.pad((xl + baWe).reshape(N * 5, NODE),
                      ((0, NP5 - N * 5), (0, 0)))
        XRp = jnp.pad((xr + bbWe + bconst[None]).reshape(N * 5, NODE),
                      ((0, NP5 - N * 5), (0, 0)))
        XLA = jnp.pad(xl.reshape(N * 5, NODE), ((0, NP5 - N * 5), (0, 0)))

        EPt = _ep(gea, p['We'])
        s0, s1, s2, s3, s4, maxes = _sc_score(row5, col5, XLp, XRp, EPt,
                                              p['att'])
        m_sc = jnp.max(maxes[:, :HEADS, :16], axis=(0, 2))
        m_in = jnp.broadcast_to(
            jnp.pad(m_sc, (0, 3))[:, None], (8, 128))
        parts = _sc_agg(row5, colp, s0, s1, s2, s3, s4, m_in, XLA)

        agg = parts[:, 0] + parts[:, 1]
        num = agg[:HEADS, :N, :]
        den = jnp.transpose(agg[HEADS, :N, :HEADS])
        out = num / (den[:, :, None] + 1e-16)
        attn = out.mean(axis=0) + p['gat_b']

        h = _layer_norm(x + attn, p['ln1_g'], p['ln1_b'])
        hm = jax.nn.selu(_mm(h, p['mlp_W1'], 2000) + p['mlp_b1'])
        hm = _layer_norm(hm, p['mlp_ln_g'], p['mlp_ln_b'])
        hm = _mm(hm, p['mlp_W2'], 2000) + p['mlp_b2']
        x = _layer_norm(h + hm, p['ln2_g'], p['ln2_b'])

        ones = jnp.ones((n, 1), x.dtype)
        cnt = jnp.maximum(jax.ops.segment_sum(ones, batch, num_segments=b), 1.0)
        s = jax.ops.segment_sum(x, batch, num_segments=b)
        mean = s / cnt
        s2 = jax.ops.segment_sum(x * x, batch, num_segments=b)
        var = s2 / cnt - mean ** 2
        std = jnp.sqrt(jax.nn.relu(var) + 1e-5)
        mx = jax.ops.segment_max(x, batch, num_segments=b)
        mn = jax.ops.segment_min(x, batch, num_segments=b)
        aggr = jnp.concatenate([mean, std, mx, mn], axis=1)
        g = jnp.concatenate([u, aggr], axis=1)
        g = jax.nn.selu(g @ p['g_W1'] + p['g_b1'])
        g = _layer_norm(g, p['g_ln_g'], p['g_ln_b'])
        u = g @ p['g_W2'] + p['g_b2']
    return u


# speedup vs baseline: 4.9386x; 1.0693x over previous
import functools
import jax
import jax.numpy as jnp
from jax import lax
from jax.experimental import pallas as pl
from jax.experimental.pallas import tpu as pltpu
from jax.experimental.pallas import tpu_sc as plsc

N = 10000
NPAD = 10112
NP5 = NPAD * 5
E = 160000
EPAD = 163840
NODE = 128
EDGE = 16
HID = 256
GLOB = 64
HEADS = 5
NC = 2
NS = 16
NW = NC * NS
EW = EPAD // NW
C = 128
NCH = EW // C
AGG = 128

_MESH = plsc.VectorSubcoreMesh(core_axis_name="c", subcore_axis_name="s",
                               num_cores=NC, num_subcores=NS)


_GDN = lax.GatherDimensionNumbers(offset_dims=(), collapsed_slice_dims=(0,),
                                  start_index_map=(0,))


def _permute(v, idx):
    return lax.gather(v, idx[:, None], _GDN, (1,),
                      mode=lax.GatherScatterMode.PROMISE_IN_BOUNDS)


def _butterfly(v, op):
    lanes = lax.iota(jnp.int32, 16)
    for sh in (8, 4, 2, 1):
        v = op(v, _permute(v, jnp.bitwise_xor(lanes, sh)))
    return v


def _mm_kernel(a_ref, w_ref, o_ref):
    o_ref[...] = jnp.dot(a_ref[...], w_ref[...],
                         preferred_element_type=jnp.float32)


@functools.partial(jax.jit, static_argnames=("bm",))
def _mm(a, w, bm):
    M, K = a.shape
    _, Nn = w.shape
    return pl.pallas_call(
        _mm_kernel,
        grid=(M // bm,),
        in_specs=[
            pl.BlockSpec((bm, K), lambda i: (i, 0)),
            pl.BlockSpec((K, Nn), lambda i: (0, 0)),
        ],
        out_specs=pl.BlockSpec((bm, Nn), lambda i: (i, 0)),
        out_shape=jax.ShapeDtypeStruct((M, Nn), jnp.float32),
    )(a, w)


def _ep_kernel(gea_ref, we_ref, o_ref):
    g = gea_ref[...]
    w = we_ref[...]
    for h in range(HEADS):
        o_ref[h] = jnp.dot(g, w[:, h * NODE:(h + 1) * NODE],
                           preferred_element_type=jnp.float32)


@jax.jit
def _ep(gea, we):
    be = 4096
    return pl.pallas_call(
        _ep_kernel,
        grid=(EPAD // be,),
        in_specs=[
            pl.BlockSpec((be, EDGE), lambda i: (i, 0)),
            pl.BlockSpec((EDGE, HEADS * NODE), lambda i: (0, 0)),
        ],
        out_specs=pl.BlockSpec((HEADS, be, NODE), lambda i: (0, i, 0)),
        out_shape=jax.ShapeDtypeStruct((HEADS, EPAD, NODE), jnp.float32),
    )(gea, we)


@functools.partial(
    pl.kernel,
    out_type=(jax.ShapeDtypeStruct((EPAD, EDGE), jnp.float32),
              jax.ShapeDtypeStruct((EPAD, EDGE), jnp.float32)),
    mesh=_MESH,
    scratch_types=[
        pltpu.VMEM((C,), jnp.int32),
        pltpu.VMEM((C,), jnp.int32),
        pltpu.VMEM((C, 128), jnp.float32),
        pltpu.VMEM((C, 128), jnp.float32),
        pltpu.VMEM((C, EDGE), jnp.float32),
        pltpu.VMEM((C, EDGE), jnp.float32),
        pltpu.VMEM((C, EDGE), jnp.float32),
        pltpu.SemaphoreType.DMA,
    ],
)
def _sc_film(row_hbm, col_hbm, nt_hbm, ea_hbm,
             ean_hbm, gea_hbm,
             idxr, idxc, abuf, bbuf, eabuf, eanbuf, geabuf, sem):
    wid = lax.axis_index("c") * NS + lax.axis_index("s")

    def chunk(ci, _):
        base = wid * EW + ci * C
        pltpu.sync_copy(row_hbm.at[pl.ds(base, C)], idxr)
        pltpu.sync_copy(col_hbm.at[pl.ds(base, C)], idxc)
        pltpu.async_copy(nt_hbm.at[idxr], abuf, sem).wait()
        pltpu.async_copy(nt_hbm.at[idxc], bbuf, sem).wait()
        pltpu.sync_copy(ea_hbm.at[pl.ds(base, C)], eabuf)

        def edge(i, _):
            ag = abuf[i, 0:16]
            ab = abuf[i, 16:32]
            bg = bbuf[i, 32:48]
            bb = bbuf[i, 48:64]
            gamma = 1.0 / (1.0 + jnp.exp(-(ag + bg)))
            ge = gamma * eabuf[i, :]
            geabuf[i, :] = ge
            eanbuf[i, :] = ge + ab + bb
            return 0

        lax.fori_loop(0, C, edge, 0)
        pltpu.sync_copy(eanbuf, ean_hbm.at[pl.ds(base, C)])
        pltpu.sync_copy(geabuf, gea_hbm.at[pl.ds(base, C)])
        return 0

    lax.fori_loop(0, NCH, chunk, 0)


@functools.partial(
    pl.kernel,
    out_type=tuple([jax.ShapeDtypeStruct((EPAD,), jnp.float32)] * HEADS
                   + [jax.ShapeDtypeStruct((NW, 8, 128), jnp.float32)]),
    mesh=_MESH,
    scratch_types=[
        pltpu.VMEM((2, C), jnp.int32),
        pltpu.VMEM((2, C), jnp.int32),
        pltpu.VMEM((C,), jnp.int32),
        pltpu.VMEM((C,), jnp.int32),
        pltpu.VMEM((2, C, NODE), jnp.float32),
        pltpu.VMEM((2, C, NODE), jnp.float32),
        pltpu.VMEM((2, C, NODE), jnp.float32),
        pltpu.VMEM((C,), jnp.float32),
        pltpu.VMEM((HEADS, NODE), jnp.float32),
        pltpu.VMEM((8, 128), jnp.float32),
        pltpu.SemaphoreType.DMA,
        pltpu.SemaphoreType.DMA,
    ],
)
def _sc_score(row5_hbm, col5_hbm, xlp_hbm, xrp_hbm, ep_hbm, att_hbm,
              sc0, sc1, sc2, sc3, sc4, max_hbm,
              idxl, idxr_, r5buf, c5buf, xlb, xrb, epb, sbuf, attb, mbuf,
              sem0, sem1):
    wid = lax.axis_index("c") * NS + lax.axis_index("s")
    score_hbms = [sc0, sc1, sc2, sc3, sc4]
    sems = [sem0, sem1]
    pltpu.sync_copy(att_hbm, attb)
    lanes = lax.iota(jnp.int32, 16)

    for h in range(HEADS):
        att_v = [attb[h, pl.ds(16 * j, 16)] for j in range(8)]

        def issue(ci, b):
            base = wid * EW + ci * C
            pltpu.sync_copy(row5_hbm.at[pl.ds(base, C)], r5buf)
            pltpu.sync_copy(col5_hbm.at[pl.ds(base, C)], c5buf)
            for j in range(C // 16):
                sl = pl.ds(16 * j, 16)
                idxl[b, sl] = r5buf[sl] + h
                idxr_[b, sl] = c5buf[sl] + h
            pltpu.async_copy(xlp_hbm.at[idxl.at[b]], xlb.at[b], sems[b])
            pltpu.async_copy(xrp_hbm.at[idxr_.at[b]], xrb.at[b], sems[b])
            pltpu.async_copy(ep_hbm.at[h, pl.ds(base, C)], epb.at[b], sems[b])

        def drain(b):
            pltpu.make_async_copy(xlp_hbm.at[idxl.at[b]], xlb.at[b],
                                  sems[b]).wait()
            pltpu.make_async_copy(xrp_hbm.at[idxr_.at[b]], xrb.at[b],
                                  sems[b]).wait()
            pltpu.make_async_copy(ep_hbm.at[h, pl.ds(0, C)], epb.at[b],
                                  sems[b]).wait()

        def compute(ci, b, m_h):
            base = wid * EW + ci * C

            def group(g, m_carry):
                def edge(k, sv):
                    i = g * 16 + k
                    acc = None
                    for j in range(8):
                        sl = pl.ds(16 * j, 16)
                        v = xlb[b, i, sl] + xrb[b, i, sl] + epb[b, i, sl]
                        v = 0.6 * v + 0.4 * jnp.abs(v)
                        t = v * att_v[j]
                        acc = t if acc is None else acc + t
                    s = _butterfly(acc, jnp.add)
                    return jnp.where(lanes == k, s, sv)

                sv = lax.fori_loop(0, 16, edge,
                                   jnp.full((16,), -3.0e38, jnp.float32))
                sbuf[pl.ds(g * 16, 16)] = sv
                return jnp.maximum(m_carry, sv)

            m_h = lax.fori_loop(0, C // 16, group, m_h)
            pltpu.sync_copy(sbuf, score_hbms[h].at[pl.ds(base, C)])
            return m_h

        issue(0, 0)

        def chunk2(ci2, m_h):
            issue(2 * ci2 + 1, 1)
            drain(0)
            m_h = compute(2 * ci2, 0, m_h)
            issue(lax.rem(2 * ci2 + 2, NCH), 0)
            drain(1)
            m_h = compute(2 * ci2 + 1, 1, m_h)
            return m_h

        m_h = lax.fori_loop(0, NCH // 2, chunk2,
                            jnp.full((16,), -3.0e38, jnp.float32))
        drain(0)
        mbuf[h, 0:16] = m_h

    pltpu.sync_copy(mbuf, max_hbm.at[wid])


@functools.partial(
    pl.kernel,
    out_type=jax.ShapeDtypeStruct((HEADS + 1, NC, NPAD, AGG), jnp.float32),
    mesh=_MESH,
    scratch_types=[
        pltpu.VMEM((C,), jnp.int32),
        pltpu.VMEM((2, C), jnp.int32),
        pltpu.VMEM((2, C), jnp.int32),
        pltpu.VMEM((2, C, AGG), jnp.float32),
        pltpu.VMEM((2, C), jnp.float32),
        pltpu.VMEM((HEADS, C), jnp.float32),
        pltpu.VMEM((8, 128), jnp.float32),
        pltpu.VMEM((64, AGG), jnp.float32),
        pltpu.VMEM_SHARED((NPAD, AGG), jnp.float32),
        pltpu.SemaphoreType.DMA,
        pltpu.SemaphoreType.DMA,
        pltpu.SemaphoreType.DMA,
        pltpu.SemaphoreType.DMA,
    ],
)
def _sc_agg(row5_hbm, col_hbm, sc0, sc1, sc2, sc3, sc4, max_hbm, xla_hbm,
            parts_hbm,
            r5buf, idxl, idxc, rbuf, sbuf, exbuf, mxbuf, zbuf,
            accum, semg0, semg1, sems0, sems1):
    score_hbms = [sc0, sc1, sc2, sc3, sc4]
    semg = [semg0, semg1]
    semsc = [sems0, sems1]
    cid = lax.axis_index("c")
    sid = lax.axis_index("s")
    wid = cid * NS + sid
    lanes = lax.iota(jnp.int32, 16)

    pltpu.sync_copy(max_hbm, mxbuf)
    m_sc = [mxbuf[h, 0:16] for h in range(HEADS)]

    def zrow(i, _):
        for j in range(AGG // 16):
            zbuf[i, pl.ds(16 * j, 16)] = jnp.zeros((16,), jnp.float32)
        return 0

    lax.fori_loop(0, 64, zrow, 0)

    rows_per_tile = NPAD // NS
    nz = rows_per_tile // 64

    def _zero_accum():
        def zchunk(i, _):
            pltpu.sync_copy(zbuf,
                            accum.at[pl.ds(sid * rows_per_tile + i * 64, 64)])
            return 0

        lax.fori_loop(0, nz, zchunk, 0)
        pltpu.sync_copy(
            zbuf.at[pl.ds(0, rows_per_tile - nz * 64)],
            accum.at[pl.ds(sid * rows_per_tile + nz * 64,
                           rows_per_tile - nz * 64)])

    def _dump(pi):
        pltpu.sync_copy(accum.at[pl.ds(sid * rows_per_tile, rows_per_tile)],
                        parts_hbm.at[pi, cid, pl.ds(sid * rows_per_tile,
                                                    rows_per_tile)])

    for h in range(HEADS):
        _zero_accum()
        plsc.subcore_barrier()
        m_h = m_sc[h]

        def issue(ci, b):
            base = wid * EW + ci * C
            pltpu.sync_copy(row5_hbm.at[pl.ds(base, C)], r5buf)
            pltpu.sync_copy(col_hbm.at[pl.ds(base, C)], idxc.at[b])
            for j in range(C // 16):
                sl = pl.ds(16 * j, 16)
                idxl[b, sl] = r5buf[sl] + h
            pltpu.async_copy(xla_hbm.at[idxl.at[b]], rbuf.at[b], semg[b])
            pltpu.async_copy(score_hbms[h].at[pl.ds(base, C)], sbuf.at[b],
                             semg[b])

        def drain_gather(b):
            pltpu.make_async_copy(xla_hbm.at[idxl.at[b]], rbuf.at[b],
                                  semg[b]).wait()
            pltpu.make_async_copy(score_hbms[h].at[pl.ds(0, C)], sbuf.at[b],
                                  semg[b]).wait()

        def drain_scatter(b):
            pltpu.make_async_copy(rbuf.at[b], accum.at[idxc.at[b]],
                                  semsc[b]).wait()

        def compute_scatter(b):
            for j in range(C // 16):
                sl = pl.ds(16 * j, 16)
                exbuf[0, sl] = jnp.exp(sbuf[b, sl] - m_h)

            def group(g, _):
                ex_g = exbuf[0, pl.ds(g * 16, 16)]

                def lane(k, _):
                    i = g * 16 + k
                    s_v = _permute(ex_g, jnp.full((16,), k, jnp.int32))
                    for j in range(AGG // 16):
                        sl = pl.ds(16 * j, 16)
                        rbuf[b, i, sl] = rbuf[b, i, sl] * s_v
                    return 0

                lax.fori_loop(0, 16, lane, 0)
                return 0

            lax.fori_loop(0, C // 16, group, 0)
            pltpu.async_copy(rbuf.at[b], accum.at[idxc.at[b]], semsc[b],
                             add=True)

        issue(0, 0)

        def chunk2(ci2, _):
            issue(2 * ci2 + 1, 1)
            drain_gather(0)
            compute_scatter(0)
            drain_scatter(0)
            issue(lax.rem(2 * ci2 + 2, NCH), 0)
            drain_gather(1)
            compute_scatter(1)
            drain_scatter(1)
            return 0

        lax.fori_loop(0, NCH // 2, chunk2, 0)
        drain_gather(0)
        plsc.subcore_barrier()
        _dump(h)
        plsc.subcore_barrier()

    _zero_accum()
    plsc.subcore_barrier()

    def zstage(i, _):
        for j in range(AGG // 16):
            rbuf[0, i, pl.ds(16 * j, 16)] = jnp.zeros((16,), jnp.float32)
        return 0

    lax.fori_loop(0, C, zstage, 0)

    def dchunk(ci, _):
        base = wid * EW + ci * C
        pltpu.sync_copy(col_hbm.at[pl.ds(base, C)], idxc.at[0])
        for h in range(HEADS):
            pltpu.sync_copy(score_hbms[h].at[pl.ds(base, C)], sbuf.at[0])
            for j in range(C // 16):
                sl = pl.ds(16 * j, 16)
                exbuf[h, sl] = jnp.exp(sbuf[0, sl] - m_sc[h])

        def group(g, _):
            ex_gs = [exbuf[h, pl.ds(g * 16, 16)] for h in range(HEADS)]

            def lane(k, _):
                i = g * 16 + k
                kk = jnp.full((16,), k, jnp.int32)
                v = jnp.zeros((16,), jnp.float32)
                for h in range(HEADS):
                    v = jnp.where(lanes == h, _permute(ex_gs[h], kk), v)
                rbuf[0, i, 0:16] = v
                return 0

            lax.fori_loop(0, 16, lane, 0)
            return 0

        lax.fori_loop(0, C // 16, group, 0)
        pltpu.sync_copy(rbuf.at[0], accum.at[idxc.at[0]], add=True)
        return 0

    lax.fori_loop(0, NCH, dchunk, 0)
    plsc.subcore_barrier()
    _dump(HEADS)


def _layer_norm(h, g, b):
    m = jnp.mean(h, axis=-1, keepdims=True)
    v = jnp.mean((h - m) ** 2, axis=-1, keepdims=True)
    return (h - m) / jnp.sqrt(v + 1e-5) * g + b


def kernel(x, edge_index, edge_attr, u, batch, params):
    row, col = edge_index[0], edge_index[1]
    n = x.shape[0]
    b = u.shape[0]
    pad = EPAD - E
    rowp = jnp.concatenate([row, jnp.zeros((pad,), jnp.int32)])
    colp = jnp.concatenate([col, jnp.full((pad,), N, jnp.int32)])
    row5 = rowp * 5
    col5 = colp * 5
    ea = jnp.concatenate([edge_attr, jnp.zeros((pad, EDGE), jnp.float32)])

    for p in params['layers']:
        Wg = p['gamma_W']
        Wb = p['beta_W']
        ug = u @ Wg[2 * NODE:]
        ub = u @ Wb[2 * NODE:]
        nW = jnp.concatenate([Wg[:NODE], Wg[NODE:2 * NODE],
                              Wb[:NODE], Wb[NODE:2 * NODE]], axis=1)
        nt = _mm(x, nW, 2000)
        ga = nt[:, :EDGE] + ug[batch] + p['gamma_b']
        gb = nt[:, EDGE:2 * EDGE]
        ba = nt[:, 2 * EDGE:3 * EDGE] + ub[batch] + p['beta_b']
        bb = nt[:, 3 * EDGE:]
        NT = jnp.pad(jnp.concatenate([ga, ba, gb, bb], axis=1),
                     ((0, NPAD - N), (0, NODE - 4 * EDGE)))

        ea, gea = _sc_film(rowp, colp, NT, ea)

        xl = _mm(x, p['Wl'], 2000)
        xr = _mm(x, p['Wr'], 2000)
        baWe = _mm(ba, p['We'], 2000)
        bbWe = _mm(bb, p['We'], 2000)
        bconst = p['beta_b'] @ p['We']
        XLp = jnp.pad((xl + baWe).reshape(N * 5, NODE),
                      ((0, NP5 - N * 5), (0, 0)))
        XRp = jnp.pad((xr + bbWe + bconst[None]).reshape(N * 5, NODE),
                      ((0, NP5 - N * 5), (0, 0)))
        XLA = jnp.pad(xl.reshape(N * 5, NODE), ((0, NP5 - N * 5), (0, 0)))

        EPt = _ep(gea, p['We'])
        s0, s1, s2, s3, s4, maxes = _sc_score(row5, col5, XLp, XRp, EPt,
                                              p['att'])
        m_sc = jnp.max(maxes[:, :HEADS, :16], axis=(0, 2))
        m_in = jnp.broadcast_to(
            jnp.pad(m_sc, (0, 3))[:, None], (8, 128))
        parts = _sc_agg(row5, colp, s0, s1, s2, s3, s4, m_in, XLA)

        agg = parts[:, 0] + parts[:, 1]
        num = agg[:HEADS, :N, :]
        den = jnp.transpose(agg[HEADS, :N, :HEADS])
        out = num / (den[:, :, None] + 1e-16)
        attn = out.mean(axis=0) + p['gat_b']

        h = _layer_norm(x + attn, p['ln1_g'], p['ln1_b'])
        hm = jax.nn.selu(_mm(h, p['mlp_W1'], 2000) + p['mlp_b1'])
        hm = _layer_norm(hm, p['mlp_ln_g'], p['mlp_ln_b'])
        hm = _mm(hm, p['mlp_W2'], 2000) + p['mlp_b2']
        x = _layer_norm(h + hm, p['ln2_g'], p['ln2_b'])

        ones = jnp.ones((n, 1), x.dtype)
        cnt = jnp.maximum(jax.ops.segment_sum(ones, batch, num_segments=b), 1.0)
        s = jax.ops.segment_sum(x, batch, num_segments=b)
        mean = s / cnt
        s2 = jax.ops.segment_sum(x * x, batch, num_segments=b)
        var = s2 / cnt - mean ** 2
        std = jnp.sqrt(jax.nn.relu(var) + 1e-5)
        mx = jax.ops.segment_max(x, batch, num_segments=b)
        mn = jax.ops.segment_min(x, batch, num_segments=b)
        aggr = jnp.concatenate([mean, std, mx, mn], axis=1)
        g = jnp.concatenate([u, aggr], axis=1)
        g = jax.nn.selu(g @ p['g_W1'] + p['g_b1'])
        g = _layer_norm(g, p['g_ln_g'], p['g_ln_b'])
        u = g @ p['g_W2'] + p['g_b2']
    return u


# speedup vs baseline: 5.2739x; 1.0679x over previous
import functools
import jax
import jax.numpy as jnp
from jax import lax
from jax.experimental import pallas as pl
from jax.experimental.pallas import tpu as pltpu
from jax.experimental.pallas import tpu_sc as plsc

N = 10000
NPAD = 10112
NP5 = NPAD * 5
E = 160000
EPAD = 163840
NODE = 128
EDGE = 16
HID = 256
GLOB = 64
HEADS = 5
NC = 2
NS = 16
NW = NC * NS
EW = EPAD // NW
C = 128
NCH = EW // C
CA = 64
NCHA = EW // CA
AGG = 128

_MESH = plsc.VectorSubcoreMesh(core_axis_name="c", subcore_axis_name="s",
                               num_cores=NC, num_subcores=NS)


_GDN = lax.GatherDimensionNumbers(offset_dims=(), collapsed_slice_dims=(0,),
                                  start_index_map=(0,))


def _permute(v, idx):
    return lax.gather(v, idx[:, None], _GDN, (1,),
                      mode=lax.GatherScatterMode.PROMISE_IN_BOUNDS)


def _butterfly(v, op):
    lanes = lax.iota(jnp.int32, 16)
    for sh in (8, 4, 2, 1):
        v = op(v, _permute(v, jnp.bitwise_xor(lanes, sh)))
    return v


def _mm_kernel(a_ref, w_ref, o_ref):
    o_ref[...] = jnp.dot(a_ref[...], w_ref[...],
                         preferred_element_type=jnp.float32)


@functools.partial(jax.jit, static_argnames=("bm",))
def _mm(a, w, bm):
    M, K = a.shape
    _, Nn = w.shape
    return pl.pallas_call(
        _mm_kernel,
        grid=(M // bm,),
        in_specs=[
            pl.BlockSpec((bm, K), lambda i: (i, 0)),
            pl.BlockSpec((K, Nn), lambda i: (0, 0)),
        ],
        out_specs=pl.BlockSpec((bm, Nn), lambda i: (i, 0)),
        out_shape=jax.ShapeDtypeStruct((M, Nn), jnp.float32),
    )(a, w)


def _ep_kernel(gea_ref, we_ref, o_ref):
    g = gea_ref[...]
    w = we_ref[...]
    for h in range(HEADS):
        o_ref[h] = jnp.dot(g, w[:, h * NODE:(h + 1) * NODE],
                           preferred_element_type=jnp.float32)


@jax.jit
def _ep(gea, we):
    be = 4096
    return pl.pallas_call(
        _ep_kernel,
        grid=(EPAD // be,),
        in_specs=[
            pl.BlockSpec((be, EDGE), lambda i: (i, 0)),
            pl.BlockSpec((EDGE, HEADS * NODE), lambda i: (0, 0)),
        ],
        out_specs=pl.BlockSpec((HEADS, be, NODE), lambda i: (0, i, 0)),
        out_shape=jax.ShapeDtypeStruct((HEADS, EPAD, NODE), jnp.float32),
    )(gea, we)


@functools.partial(
    pl.kernel,
    out_type=(jax.ShapeDtypeStruct((EPAD, EDGE), jnp.float32),
              jax.ShapeDtypeStruct((EPAD, EDGE), jnp.float32)),
    mesh=_MESH,
    scratch_types=[
        pltpu.VMEM((2, CA), jnp.int32),
        pltpu.VMEM((2, CA), jnp.int32),
        pltpu.VMEM((2, CA, 128), jnp.float32),
        pltpu.VMEM((2, CA, 128), jnp.float32),
        pltpu.VMEM((2, CA, EDGE), jnp.float32),
        pltpu.VMEM((2, CA, EDGE), jnp.float32),
        pltpu.VMEM((2, CA, EDGE), jnp.float32),
        pltpu.SemaphoreType.DMA,
        pltpu.SemaphoreType.DMA,
        pltpu.SemaphoreType.DMA,
        pltpu.SemaphoreType.DMA,
    ],
)
def _sc_film(row_hbm, col_hbm, nt_hbm, ea_hbm,
             ean_hbm, gea_hbm,
             idxr, idxc, abuf, bbuf, eabuf, eanbuf, geabuf,
             semg0, semg1, semw0, semw1):
    wid = lax.axis_index("c") * NS + lax.axis_index("s")
    semg = [semg0, semg1]
    semw = [semw0, semw1]

    def issue(ci, b):
        base = wid * EW + ci * CA
        pltpu.sync_copy(row_hbm.at[pl.ds(base, CA)], idxr.at[b])
        pltpu.sync_copy(col_hbm.at[pl.ds(base, CA)], idxc.at[b])
        pltpu.async_copy(nt_hbm.at[idxr.at[b]], abuf.at[b], semg[b])
        pltpu.async_copy(nt_hbm.at[idxc.at[b]], bbuf.at[b], semg[b])
        pltpu.async_copy(ea_hbm.at[pl.ds(base, CA)], eabuf.at[b], semg[b])

    def drain_g(b):
        pltpu.make_async_copy(nt_hbm.at[idxr.at[b]], abuf.at[b],
                              semg[b]).wait()
        pltpu.make_async_copy(nt_hbm.at[idxc.at[b]], bbuf.at[b],
                              semg[b]).wait()
        pltpu.make_async_copy(ea_hbm.at[pl.ds(0, CA)], eabuf.at[b],
                              semg[b]).wait()

    def drain_w(b):
        pltpu.make_async_copy(eanbuf.at[b], ean_hbm.at[pl.ds(0, CA)],
                              semw[b]).wait()
        pltpu.make_async_copy(geabuf.at[b], gea_hbm.at[pl.ds(0, CA)],
                              semw[b]).wait()

    def compute(ci, b):
        base = wid * EW + ci * CA

        def edge(i, _):
            ag = abuf[b, i, 0:16]
            ab = abuf[b, i, 16:32]
            bg = bbuf[b, i, 32:48]
            bb = bbuf[b, i, 48:64]
            gamma = 1.0 / (1.0 + jnp.exp(-(ag + bg)))
            ge = gamma * eabuf[b, i, :]
            geabuf[b, i, :] = ge
            eanbuf[b, i, :] = ge + ab + bb
            return 0

        lax.fori_loop(0, CA, edge, 0)
        pltpu.async_copy(eanbuf.at[b], ean_hbm.at[pl.ds(base, CA)], semw[b])
        pltpu.async_copy(geabuf.at[b], gea_hbm.at[pl.ds(base, CA)], semw[b])

    issue(0, 0)

    def chunk2(ci2, _):
        issue(2 * ci2 + 1, 1)
        drain_g(0)

        @pl.when(ci2 > 0)
        def _():
            drain_w(0)

        compute(2 * ci2, 0)
        issue(lax.rem(2 * ci2 + 2, NCHA), 0)
        drain_g(1)

        @pl.when(ci2 > 0)
        def _():
            drain_w(1)

        compute(2 * ci2 + 1, 1)
        return 0

    lax.fori_loop(0, NCHA // 2, chunk2, 0)
    drain_g(0)
    drain_w(0)
    drain_w(1)


@functools.partial(
    pl.kernel,
    out_type=tuple([jax.ShapeDtypeStruct((EPAD,), jnp.float32)] * HEADS
                   + [jax.ShapeDtypeStruct((NW, 8, 128), jnp.float32)]),
    mesh=_MESH,
    scratch_types=[
        pltpu.VMEM((2, C), jnp.int32),
        pltpu.VMEM((2, C), jnp.int32),
        pltpu.VMEM((C,), jnp.int32),
        pltpu.VMEM((C,), jnp.int32),
        pltpu.VMEM((2, C, NODE), jnp.float32),
        pltpu.VMEM((2, C, NODE), jnp.float32),
        pltpu.VMEM((2, C, NODE), jnp.float32),
        pltpu.VMEM((C,), jnp.float32),
        pltpu.VMEM((HEADS, NODE), jnp.float32),
        pltpu.VMEM((8, 128), jnp.float32),
        pltpu.SemaphoreType.DMA,
        pltpu.SemaphoreType.DMA,
    ],
)
def _sc_score(row5_hbm, col5_hbm, xlp_hbm, xrp_hbm, ep_hbm, att_hbm,
              sc0, sc1, sc2, sc3, sc4, max_hbm,
              idxl, idxr_, r5buf, c5buf, xlb, xrb, epb, sbuf, attb, mbuf,
              sem0, sem1):
    wid = lax.axis_index("c") * NS + lax.axis_index("s")
    score_hbms = [sc0, sc1, sc2, sc3, sc4]
    sems = [sem0, sem1]
    pltpu.sync_copy(att_hbm, attb)
    lanes = lax.iota(jnp.int32, 16)

    for h in range(HEADS):
        att_v = [attb[h, pl.ds(16 * j, 16)] for j in range(8)]

        def issue(ci, b):
            base = wid * EW + ci * C
            pltpu.sync_copy(row5_hbm.at[pl.ds(base, C)], r5buf)
            pltpu.sync_copy(col5_hbm.at[pl.ds(base, C)], c5buf)
            for j in range(C // 16):
                sl = pl.ds(16 * j, 16)
                idxl[b, sl] = r5buf[sl] + h
                idxr_[b, sl] = c5buf[sl] + h
            pltpu.async_copy(xlp_hbm.at[idxl.at[b]], xlb.at[b], sems[b])
            pltpu.async_copy(xrp_hbm.at[idxr_.at[b]], xrb.at[b], sems[b])
            pltpu.async_copy(ep_hbm.at[h, pl.ds(base, C)], epb.at[b], sems[b])

        def drain(b):
            pltpu.make_async_copy(xlp_hbm.at[idxl.at[b]], xlb.at[b],
                                  sems[b]).wait()
            pltpu.make_async_copy(xrp_hbm.at[idxr_.at[b]], xrb.at[b],
                                  sems[b]).wait()
            pltpu.make_async_copy(ep_hbm.at[h, pl.ds(0, C)], epb.at[b],
                                  sems[b]).wait()

        def compute(ci, b, m_h):
            base = wid * EW + ci * C

            def group(g, m_carry):
                def edge(k, sv):
                    i = g * 16 + k
                    acc = None
                    for j in range(8):
                        sl = pl.ds(16 * j, 16)
                        v = xlb[b, i, sl] + xrb[b, i, sl] + epb[b, i, sl]
                        v = 0.6 * v + 0.4 * jnp.abs(v)
                        t = v * att_v[j]
                        acc = t if acc is None else acc + t
                    s = _butterfly(acc, jnp.add)
                    return jnp.where(lanes == k, s, sv)

                sv = lax.fori_loop(0, 16, edge,
                                   jnp.full((16,), -3.0e38, jnp.float32))
                sbuf[pl.ds(g * 16, 16)] = sv
                return jnp.maximum(m_carry, sv)

            m_h = lax.fori_loop(0, C // 16, group, m_h)
            pltpu.sync_copy(sbuf, score_hbms[h].at[pl.ds(base, C)])
            return m_h

        issue(0, 0)

        def chunk2(ci2, m_h):
            issue(2 * ci2 + 1, 1)
            drain(0)
            m_h = compute(2 * ci2, 0, m_h)
            issue(lax.rem(2 * ci2 + 2, NCH), 0)
            drain(1)
            m_h = compute(2 * ci2 + 1, 1, m_h)
            return m_h

        m_h = lax.fori_loop(0, NCH // 2, chunk2,
                            jnp.full((16,), -3.0e38, jnp.float32))
        drain(0)
        mbuf[h, 0:16] = m_h

    pltpu.sync_copy(mbuf, max_hbm.at[wid])


@functools.partial(
    pl.kernel,
    out_type=jax.ShapeDtypeStruct((HEADS + 1, NC, NPAD, AGG), jnp.float32),
    mesh=_MESH,
    scratch_types=[
        pltpu.VMEM((C,), jnp.int32),
        pltpu.VMEM((2, C), jnp.int32),
        pltpu.VMEM((2, C), jnp.int32),
        pltpu.VMEM((2, C, AGG), jnp.float32),
        pltpu.VMEM((2, C), jnp.float32),
        pltpu.VMEM((HEADS, C), jnp.float32),
        pltpu.VMEM((8, 128), jnp.float32),
        pltpu.VMEM((64, AGG), jnp.float32),
        pltpu.VMEM_SHARED((NPAD, AGG), jnp.float32),
        pltpu.SemaphoreType.DMA,
        pltpu.SemaphoreType.DMA,
        pltpu.SemaphoreType.DMA,
        pltpu.SemaphoreType.DMA,
    ],
)
def _sc_agg(row5_hbm, col_hbm, sc0, sc1, sc2, sc3, sc4, max_hbm, xla_hbm,
            parts_hbm,
            r5buf, idxl, idxc, rbuf, sbuf, exbuf, mxbuf, zbuf,
            accum, semg0, semg1, sems0, sems1):
    score_hbms = [sc0, sc1, sc2, sc3, sc4]
    semg = [semg0, semg1]
    semsc = [sems0, sems1]
    cid = lax.axis_index("c")
    sid = lax.axis_index("s")
    wid = cid * NS + sid
    lanes = lax.iota(jnp.int32, 16)

    pltpu.sync_copy(max_hbm, mxbuf)
    m_sc = [mxbuf[h, 0:16] for h in range(HEADS)]

    def zrow(i, _):
        for j in range(AGG // 16):
            zbuf[i, pl.ds(16 * j, 16)] = jnp.zeros((16,), jnp.float32)
        return 0

    lax.fori_loop(0, 64, zrow, 0)

    rows_per_tile = NPAD // NS
    nz = rows_per_tile // 64

    def _zero_accum():
        def zchunk(i, _):
            pltpu.sync_copy(zbuf,
                            accum.at[pl.ds(sid * rows_per_tile + i * 64, 64)])
            return 0

        lax.fori_loop(0, nz, zchunk, 0)
        pltpu.sync_copy(
            zbuf.at[pl.ds(0, rows_per_tile - nz * 64)],
            accum.at[pl.ds(sid * rows_per_tile + nz * 64,
                           rows_per_tile - nz * 64)])

    def _dump(pi):
        pltpu.sync_copy(accum.at[pl.ds(sid * rows_per_tile, rows_per_tile)],
                        parts_hbm.at[pi, cid, pl.ds(sid * rows_per_tile,
                                                    rows_per_tile)])

    for h in range(HEADS):
        _zero_accum()
        plsc.subcore_barrier()
        m_h = m_sc[h]

        def issue(ci, b):
            base = wid * EW + ci * C
            pltpu.sync_copy(row5_hbm.at[pl.ds(base, C)], r5buf)
            pltpu.sync_copy(col_hbm.at[pl.ds(base, C)], idxc.at[b])
            for j in range(C // 16):
                sl = pl.ds(16 * j, 16)
                idxl[b, sl] = r5buf[sl] + h
            pltpu.async_copy(xla_hbm.at[idxl.at[b]], rbuf.at[b], semg[b])
            pltpu.async_copy(score_hbms[h].at[pl.ds(base, C)], sbuf.at[b],
                             semg[b])

        def drain_gather(b):
            pltpu.make_async_copy(xla_hbm.at[idxl.at[b]], rbuf.at[b],
                                  semg[b]).wait()
            pltpu.make_async_copy(score_hbms[h].at[pl.ds(0, C)], sbuf.at[b],
                                  semg[b]).wait()

        def drain_scatter(b):
            pltpu.make_async_copy(rbuf.at[b], accum.at[idxc.at[b]],
                                  semsc[b]).wait()

        def compute_scatter(b):
            for j in range(C // 16):
                sl = pl.ds(16 * j, 16)
                exbuf[0, sl] = jnp.exp(sbuf[b, sl] - m_h)

            def group(g, _):
                ex_g = exbuf[0, pl.ds(g * 16, 16)]

                def lane(k, _):
                    i = g * 16 + k
                    s_v = _permute(ex_g, jnp.full((16,), k, jnp.int32))
                    for j in range(AGG // 16):
                        sl = pl.ds(16 * j, 16)
                        rbuf[b, i, sl] = rbuf[b, i, sl] * s_v
                    return 0

                lax.fori_loop(0, 16, lane, 0)
                return 0

            lax.fori_loop(0, C // 16, group, 0)
            pltpu.async_copy(rbuf.at[b], accum.at[idxc.at[b]], semsc[b],
                             add=True)

        issue(0, 0)

        def chunk2(ci2, _):
            issue(2 * ci2 + 1, 1)
            drain_gather(0)
            compute_scatter(0)
            drain_scatter(0)
            issue(lax.rem(2 * ci2 + 2, NCH), 0)
            drain_gather(1)
            compute_scatter(1)
            drain_scatter(1)
            return 0

        lax.fori_loop(0, NCH // 2, chunk2, 0)
        drain_gather(0)
        plsc.subcore_barrier()
        _dump(h)
        plsc.subcore_barrier()

    _zero_accum()
    plsc.subcore_barrier()

    def zstage(i, _):
        for j in range(AGG // 16):
            rbuf[0, i, pl.ds(16 * j, 16)] = jnp.zeros((16,), jnp.float32)
        return 0

    lax.fori_loop(0, C, zstage, 0)

    def dchunk(ci, _):
        base = wid * EW + ci * C
        pltpu.sync_copy(col_hbm.at[pl.ds(base, C)], idxc.at[0])
        for h in range(HEADS):
            pltpu.sync_copy(score_hbms[h].at[pl.ds(base, C)], sbuf.at[0])
            for j in range(C // 16):
                sl = pl.ds(16 * j, 16)
                exbuf[h, sl] = jnp.exp(sbuf[0, sl] - m_sc[h])

        def group(g, _):
            ex_gs = [exbuf[h, pl.ds(g * 16, 16)] for h in range(HEADS)]

            def lane(k, _):
                i = g * 16 + k
                kk = jnp.full((16,), k, jnp.int32)
                v = jnp.zeros((16,), jnp.float32)
                for h in range(HEADS):
                    v = jnp.where(lanes == h, _permute(ex_gs[h], kk), v)
                rbuf[0, i, 0:16] = v
                return 0

            lax.fori_loop(0, 16, lane, 0)
            return 0

        lax.fori_loop(0, C // 16, group, 0)
        pltpu.sync_copy(rbuf.at[0], accum.at[idxc.at[0]], add=True)
        return 0

    lax.fori_loop(0, NCH, dchunk, 0)
    plsc.subcore_barrier()
    _dump(HEADS)


def _layer_norm(h, g, b):
    m = jnp.mean(h, axis=-1, keepdims=True)
    v = jnp.mean((h - m) ** 2, axis=-1, keepdims=True)
    return (h - m) / jnp.sqrt(v + 1e-5) * g + b


def kernel(x, edge_index, edge_attr, u, batch, params):
    row, col = edge_index[0], edge_index[1]
    n = x.shape[0]
    b = u.shape[0]
    pad = EPAD - E
    rowp = jnp.concatenate([row, jnp.zeros((pad,), jnp.int32)])
    colp = jnp.concatenate([col, jnp.full((pad,), N, jnp.int32)])
    row5 = rowp * 5
    col5 = colp * 5
    ea = jnp.concatenate([edge_attr, jnp.zeros((pad, EDGE), jnp.float32)])

    for p in params['layers']:
        Wg = p['gamma_W']
        Wb = p['beta_W']
        ug = u @ Wg[2 * NODE:]
        ub = u @ Wb[2 * NODE:]
        nW = jnp.concatenate([Wg[:NODE], Wg[NODE:2 * NODE],
                              Wb[:NODE], Wb[NODE:2 * NODE]], axis=1)
        nt = _mm(x, nW, 2000)
        ga = nt[:, :EDGE] + ug[batch] + p['gamma_b']
        gb = nt[:, EDGE:2 * EDGE]
        ba = nt[:, 2 * EDGE:3 * EDGE] + ub[batch] + p['beta_b']
        bb = nt[:, 3 * EDGE:]
        NT = jnp.pad(jnp.concatenate([ga, ba, gb, bb], axis=1),
                     ((0, NPAD - N), (0, NODE - 4 * EDGE)))

        ea, gea = _sc_film(rowp, colp, NT, ea)

        xl = _mm(x, p['Wl'], 2000)
        xr = _mm(x, p['Wr'], 2000)
        baWe = _mm(ba, p['We'], 2000)
        bbWe = _mm(bb, p['We'], 2000)
        bconst = p['beta_b'] @ p['We']
        XLp = jnp.pad((xl + baWe).reshape(N * 5, NODE),
                      ((0, NP5 - N * 5), (0, 0)))
        XRp = jnp.pad((xr + bbWe + bconst[None]).reshape(N * 5, NODE),
                      ((0, NP5 - N * 5), (0, 0)))
        XLA = jnp.pad(xl.reshape(N * 5, NODE), ((0, NP5 - N * 5), (0, 0)))

        EPt = _ep(gea, p['We'])
        s0, s1, s2, s3, s4, maxes = _sc_score(row5, col5, XLp, XRp, EPt,
                                              p['att'])
        m_sc = jnp.max(maxes[:, :HEADS, :16], axis=(0, 2))
        m_in = jnp.broadcast_to(
            jnp.pad(m_sc, (0, 3))[:, None], (8, 128))
        parts = _sc_agg(row5, colp, s0, s1, s2, s3, s4, m_in, XLA)

        agg = parts[:, 0] + parts[:, 1]
        num = agg[:HEADS, :N, :]
        den = jnp.transpose(agg[HEADS, :N, :HEADS])
        out = num / (den[:, :, None] + 1e-16)
        attn = out.mean(axis=0) + p['gat_b']

        h = _layer_norm(x + attn, p['ln1_g'], p['ln1_b'])
        hm = jax.nn.selu(_mm(h, p['mlp_W1'], 2000) + p['mlp_b1'])
        hm = _layer_norm(hm, p['mlp_ln_g'], p['mlp_ln_b'])
        hm = _mm(hm, p['mlp_W2'], 2000) + p['mlp_b2']
        x = _layer_norm(h + hm, p['ln2_g'], p['ln2_b'])

        ones = jnp.ones((n, 1), x.dtype)
        cnt = jnp.maximum(jax.ops.segment_sum(ones, batch, num_segments=b), 1.0)
        s = jax.ops.segment_sum(x, batch, num_segments=b)
        mean = s / cnt
        s2 = jax.ops.segment_sum(x * x, batch, num_segments=b)
        var = s2 / cnt - mean ** 2
        std = jnp.sqrt(jax.nn.relu(var) + 1e-5)
        mx = jax.ops.segment_max(x, batch, num_segments=b)
        mn = jax.ops.segment_min(x, batch, num_segments=b)
        aggr = jnp.concatenate([mean, std, mx, mn], axis=1)
        g = jnp.concatenate([u, aggr], axis=1)
        g = jax.nn.selu(g @ p['g_W1'] + p['g_b1'])
        g = _layer_norm(g, p['g_ln_g'], p['g_ln_b'])
        u = g @ p['g_W2'] + p['g_b2']
    return u
